# baseline (speedup 1.0000x reference)
import functools

import jax
import jax.numpy as jnp
import numpy as np
from jax import lax
from jax.experimental import pallas as pl
from jax.experimental.pallas import tpu as pltpu
from jax.experimental.pallas import tpu_sc as plsc

VOCAB = 100000
D = 64
MAXLEN = 200
NG1 = 21
MAX_G = 3
BATCH = 4096
SEQLEN = 200
N = BATCH * SEQLEN


def _pe_table():
    pe = np.zeros((MAXLEN, D), dtype=np.float32)
    position = np.arange(MAXLEN, dtype=np.float32)[:, None]
    div_term = np.exp(np.arange(0, D, 2, dtype=np.float32) * (-np.log(10000.0) / D))
    pe[:, 0::2] = np.sin(position * div_term)
    pe[:, 1::2] = np.cos(position * div_term)
    return pe


_PE = _pe_table()


_U = 1000


def _fuse_body(tok2_ref, gid2_ref, gtab2_ref, out_ref):
    gid2 = gid2_ref[...]
    iota = lax.broadcasted_iota(jnp.int32, (_U, 2 * NG1), 1)
    counts = jnp.zeros((_U, 2 * NG1), jnp.float32)
    for g in range(2 * MAX_G):
        gid_g = lax.slice(gid2, (0, g), (_U, g + 1))
        off = NG1 if g >= MAX_G else 0
        counts = counts + ((gid_g + off) == iota).astype(jnp.float32)
    gavg = lax.dot_general(
        counts, gtab2_ref[...], (((1,), (0,)), ((), ())),
        preferred_element_type=jnp.float32,
    )
    out_ref[...] = tok2_ref[...] + gavg * (1.0 / MAX_G)


def _build_fused(token_table, genre_table, token_to_genres):
    tok2 = token_table.reshape(VOCAB // 2, 2 * D)
    gid2 = token_to_genres.reshape(VOCAB // 2, 2 * MAX_G)
    zero = jnp.zeros_like(genre_table)
    gtab2 = jnp.concatenate(
        [jnp.concatenate([genre_table, zero], axis=1),
         jnp.concatenate([zero, genre_table], axis=1)], axis=0)
    return pl.pallas_call(
        _fuse_body,
        grid=(VOCAB // 2 // _U,),
        in_specs=[
            pl.BlockSpec((_U, 2 * D), lambda i: (i, 0)),
            pl.BlockSpec((_U, 2 * MAX_G), lambda i: (i, 0)),
            pl.BlockSpec((2 * NG1, 2 * D), lambda i: (0, 0)),
        ],
        out_specs=pl.BlockSpec((_U, 2 * D), lambda i: (i, 0)),
        out_shape=jax.ShapeDtypeStruct((VOCAB // 2, 2 * D), jnp.float32),
    )(tok2, gid2, gtab2)


_NW = 32
_TOK_CH = 200
_R32 = 2 * _TOK_CH
_PER_W = N // _NW
_NCH = _PER_W // _TOK_CH
_IW = 100
_GPC = _R32 // _IW


def _gather_pe_body(fused_hbm, idxr_hbm, pe_hbm, out_hbm, idx_v, rows_v, pe_v, sem):
    wid = lax.axis_index("s") * 2 + lax.axis_index("c")
    base = wid * _PER_W * 2
    irow0 = wid * (_PER_W * 2 // _IW)
    pltpu.sync_copy(pe_hbm, pe_v)

    @pl.loop(0, _NCH)
    def _(c):
        pltpu.sync_copy(idxr_hbm.at[pl.ds(irow0 + c * _GPC, _GPC)], idx_v)
        cps = [
            pltpu.async_copy(
                fused_hbm.at[idx_v.at[k]],
                rows_v.at[pl.ds(k * _IW, _IW)], sem)
            for k in range(_GPC)
        ]
        for cp in cps:
            cp.wait()

        @pl.loop(0, _R32)
        def _(j):
            for s in range(32 // 16):
                sl = pl.ds(s * 16, 16)
                rows_v[j, sl] = rows_v[j, sl] + pe_v[j, sl]

        pltpu.sync_copy(rows_v, out_hbm.at[pl.ds(base + c * _R32, _R32)])


@functools.cache
def _gather_pe():
    mesh = plsc.VectorSubcoreMesh(core_axis_name="c", subcore_axis_name="s")
    return pl.kernel(
        _gather_pe_body,
        out_type=jax.ShapeDtypeStruct((2 * N, 32), jnp.float32),
        mesh=mesh,
        scratch_types=[
            pltpu.VMEM((_GPC, _IW), jnp.int32),
            pltpu.VMEM((_R32, 32), jnp.float32),
            pltpu.VMEM((2 * MAXLEN, 32), jnp.float32),
            pltpu.SemaphoreType.DMA,
        ],
        compiler_params=pltpu.CompilerParams(use_tc_tiling_on_sc=False),
    )


def kernel(sequence, token_table, genre_table, token_to_genres):
    fused = _build_fused(token_table, genre_table, token_to_genres)
    fused32 = fused.reshape(2 * VOCAB, 32)
    s = sequence.reshape(-1)
    idx2 = jnp.stack([s * 2, s * 2 + 1], axis=-1).reshape(2 * N // _IW, _IW)
    pe32 = jnp.asarray(_PE.reshape(2 * MAXLEN, 32))
    out = _gather_pe()(fused32, idx2, pe32)
    return out.reshape(BATCH, SEQLEN, D)


# speedup vs baseline: 2.0244x; 2.0244x over previous
import functools

import jax
import jax.numpy as jnp
import numpy as np
from jax import lax
from jax.experimental import pallas as pl
from jax.experimental.pallas import tpu as pltpu
from jax.experimental.pallas import tpu_sc as plsc

VOCAB = 100000
D = 64
MAXLEN = 200
NG1 = 21
MAX_G = 3
BATCH = 4096
SEQLEN = 200
N = BATCH * SEQLEN


def _pe_table():
    pe = np.zeros((MAXLEN, D), dtype=np.float32)
    position = np.arange(MAXLEN, dtype=np.float32)[:, None]
    div_term = np.exp(np.arange(0, D, 2, dtype=np.float32) * (-np.log(10000.0) / D))
    pe[:, 0::2] = np.sin(position * div_term)
    pe[:, 1::2] = np.cos(position * div_term)
    return pe


_PE = _pe_table()


_U = 1000


def _fuse_body(tok2_ref, gid2_ref, gtab2_ref, out_ref):
    gid2 = gid2_ref[...]
    iota = lax.broadcasted_iota(jnp.int32, (_U, 2 * NG1), 1)
    counts = jnp.zeros((_U, 2 * NG1), jnp.float32)
    for g in range(2 * MAX_G):
        gid_g = lax.slice(gid2, (0, g), (_U, g + 1))
        off = NG1 if g >= MAX_G else 0
        counts = counts + ((gid_g + off) == iota).astype(jnp.float32)
    gavg = lax.dot_general(
        counts, gtab2_ref[...], (((1,), (0,)), ((), ())),
        preferred_element_type=jnp.float32,
    )
    out_ref[...] = tok2_ref[...] + gavg * (1.0 / MAX_G)


def _build_fused(token_table, genre_table, token_to_genres):
    tok2 = token_table.reshape(VOCAB // 2, 2 * D)
    gid2 = token_to_genres.reshape(VOCAB // 2, 2 * MAX_G)
    zero = jnp.zeros_like(genre_table)
    gtab2 = jnp.concatenate(
        [jnp.concatenate([genre_table, zero], axis=1),
         jnp.concatenate([zero, genre_table], axis=1)], axis=0)
    return pl.pallas_call(
        _fuse_body,
        grid=(VOCAB // 2 // _U,),
        in_specs=[
            pl.BlockSpec((_U, 2 * D), lambda i: (i, 0)),
            pl.BlockSpec((_U, 2 * MAX_G), lambda i: (i, 0)),
            pl.BlockSpec((2 * NG1, 2 * D), lambda i: (0, 0)),
        ],
        out_specs=pl.BlockSpec((_U, 2 * D), lambda i: (i, 0)),
        out_shape=jax.ShapeDtypeStruct((VOCAB // 2, 2 * D), jnp.float32),
    )(tok2, gid2, gtab2)


_NW = 32
_BPW = BATCH // _NW
_CH = SEQLEN
_IW0 = 104
_IW1 = 96


def _gather_pe_body(fused_hbm, seq_hbm, pe_hbm, out_hbm, idx_v, rows_v, pe_v, sem):
    wid = lax.axis_index("s") * 2 + lax.axis_index("c")
    b0 = wid * _BPW
    pltpu.sync_copy(pe_hbm, pe_v)

    @pl.loop(0, _BPW)
    def _(c):
        b = b0 + c
        pltpu.sync_copy(seq_hbm.at[b], idx_v)
        cp0 = pltpu.async_copy(
            fused_hbm.at[idx_v.at[pl.ds(0, _IW0)]],
            rows_v.at[pl.ds(0, _IW0)], sem)
        cp1 = pltpu.async_copy(
            fused_hbm.at[idx_v.at[pl.ds(_IW0, _IW1)]],
            rows_v.at[pl.ds(_IW0, _IW1)], sem)
        cp0.wait()
        cp1.wait()

        @pl.loop(0, _CH)
        def _(j):
            for s in range(D // 16):
                sl = pl.ds(s * 16, 16)
                rows_v[j, sl] = rows_v[j, sl] + pe_v[j, sl]

        pltpu.sync_copy(rows_v, out_hbm.at[b])


@functools.cache
def _gather_pe():
    mesh = plsc.VectorSubcoreMesh(core_axis_name="c", subcore_axis_name="s")
    return pl.kernel(
        _gather_pe_body,
        out_type=jax.ShapeDtypeStruct((BATCH, SEQLEN, D), jnp.float32),
        mesh=mesh,
        scratch_types=[
            pltpu.VMEM((_CH,), jnp.int32),
            pltpu.VMEM((_CH, D), jnp.float32),
            pltpu.VMEM((MAXLEN, D), jnp.float32),
            pltpu.SemaphoreType.DMA,
        ],
        compiler_params=pltpu.CompilerParams(use_tc_tiling_on_sc=False),
    )


def kernel(sequence, token_table, genre_table, token_to_genres):
    fused = _build_fused(token_table, genre_table, token_to_genres)
    fused2d = fused.reshape(VOCAB, D)
    pe = jnp.asarray(_PE)
    return _gather_pe()(fused2d, sequence, pe)


# speedup vs baseline: 2.3801x; 1.1757x over previous
import functools

import jax
import jax.numpy as jnp
import numpy as np
from jax import lax
from jax.experimental import pallas as pl
from jax.experimental.pallas import tpu as pltpu
from jax.experimental.pallas import tpu_sc as plsc

VOCAB = 100000
D = 64
MAXLEN = 200
NG1 = 21
MAX_G = 3
BATCH = 4096
SEQLEN = 200
N = BATCH * SEQLEN


def _pe_table():
    pe = np.zeros((MAXLEN, D), dtype=np.float32)
    position = np.arange(MAXLEN, dtype=np.float32)[:, None]
    div_term = np.exp(np.arange(0, D, 2, dtype=np.float32) * (-np.log(10000.0) / D))
    pe[:, 0::2] = np.sin(position * div_term)
    pe[:, 1::2] = np.cos(position * div_term)
    return pe


_PE = _pe_table()


_R = 2000


def _fuse_body(tok_ref, gid_ref, gtab_ref, out_ref):
    gids = gid_ref[...]
    iota = lax.broadcasted_iota(jnp.int32, (_R, NG1), 1)
    counts = jnp.zeros((_R, NG1), jnp.float32)
    for g in range(MAX_G):
        gid_g = lax.slice(gids, (0, g), (_R, g + 1))
        counts = counts + (gid_g == iota).astype(jnp.float32)
    gavg = lax.dot_general(
        counts, gtab_ref[...], (((1,), (0,)), ((), ())),
        preferred_element_type=jnp.float32,
    )
    out_ref[...] = tok_ref[...] + gavg * (1.0 / MAX_G)


def _build_fused(token_table, genre_table, token_to_genres):
    return pl.pallas_call(
        _fuse_body,
        grid=(VOCAB // _R,),
        in_specs=[
            pl.BlockSpec((_R, D), lambda i: (i, 0)),
            pl.BlockSpec((_R, MAX_G), lambda i: (i, 0)),
            pl.BlockSpec((NG1, D), lambda i: (0, 0)),
        ],
        out_specs=pl.BlockSpec((_R, D), lambda i: (i, 0)),
        out_shape=jax.ShapeDtypeStruct((VOCAB, D), jnp.float32),
    )(token_table, token_to_genres, genre_table)


_NW = 32
_BPW = BATCH // _NW
_CH = SEQLEN
_IW0 = 104
_IW1 = 96


def _gather_pe_body(fused_hbm, seq_hbm, pe_hbm, out_hbm,
                    idx_v, rows_v, pe_v, gsems, wsems):
    wid = lax.axis_index("s") * 2 + lax.axis_index("c")
    b0 = wid * _BPW
    pltpu.sync_copy(pe_hbm, pe_v)

    def load_idx(buf, c):
        pltpu.sync_copy(seq_hbm.at[b0 + c], idx_v.at[buf])

    def start_gathers(buf, c):
        pltpu.async_copy(
            fused_hbm.at[idx_v.at[buf, pl.ds(0, _IW0)]],
            rows_v.at[buf, pl.ds(0, _IW0)], gsems.at[buf])
        pltpu.async_copy(
            fused_hbm.at[idx_v.at[buf, pl.ds(_IW0, _IW1)]],
            rows_v.at[buf, pl.ds(_IW0, _IW1)], gsems.at[buf])

    def wait_gathers(buf):
        pltpu.make_async_copy(
            out_hbm.at[b0], rows_v.at[buf], gsems.at[buf]).wait()

    def add_pe(buf):
        @pl.loop(0, _CH)
        def _(j):
            for s in range(D // 16):
                sl = pl.ds(s * 16, 16)
                rows_v[buf, j, sl] = rows_v[buf, j, sl] + pe_v[j, sl]

    def start_write(buf, c):
        pltpu.async_copy(rows_v.at[buf], out_hbm.at[b0 + c], wsems.at[buf])

    def wait_write(buf):
        pltpu.make_async_copy(
            rows_v.at[buf], out_hbm.at[b0], wsems.at[buf]).wait()

    load_idx(0, 0)
    start_gathers(0, 0)
    load_idx(1, 1)
    start_gathers(1, 1)

    @pl.loop(0, _BPW - 2, step=2)
    def _(cc):
        for buf in range(2):
            wait_gathers(buf)
            add_pe(buf)
            start_write(buf, cc + buf)
        for buf in range(2):
            load_idx(buf, cc + 2 + buf)
            wait_write(buf)
            start_gathers(buf, cc + 2 + buf)

    for buf in range(2):
        wait_gathers(buf)
        add_pe(buf)
        start_write(buf, _BPW - 2 + buf)
    for buf in range(2):
        wait_write(buf)


@functools.cache
def _gather_pe():
    mesh = plsc.VectorSubcoreMesh(core_axis_name="c", subcore_axis_name="s")
    return pl.kernel(
        _gather_pe_body,
        out_type=jax.ShapeDtypeStruct((BATCH, SEQLEN, D), jnp.float32),
        mesh=mesh,
        scratch_types=[
            pltpu.VMEM((2, _CH), jnp.int32),
            pltpu.VMEM((2, _CH, D), jnp.float32),
            pltpu.VMEM((MAXLEN, D), jnp.float32),
            pltpu.SemaphoreType.DMA((2,)),
            pltpu.SemaphoreType.DMA((2,)),
        ],
        compiler_params=pltpu.CompilerParams(use_tc_tiling_on_sc=False),
    )


def kernel(sequence, token_table, genre_table, token_to_genres):
    fused = _build_fused(token_table, genre_table, token_to_genres)
    pe = jnp.asarray(_PE)
    return _gather_pe()(fused, sequence, pe)
